# Initial kernel scaffold; baseline (speedup 1.0000x reference)
import jax
import jax.numpy as jnp
from jax.experimental import pallas as pl


def kernel(x, edge_index, W1, b1, W2, b2):
    raise NotImplementedError("write your pallas kernel here")


# speedup vs baseline: 16.1684x; 16.1684x over previous
import functools

import jax
import jax.numpy as jnp
from jax import lax
from jax.experimental import pallas as pl
from jax.experimental.pallas import tpu as pltpu
from jax.experimental.pallas import tpu_sc as plsc

N_NODES = 50000
F_IN = 1433
HID = 16
C_OUT = 7

NW = 32
CHUNK = 128
BATCH = 40
N_PAD = 50048
ROWS_PER_TILE = N_PAD // 16


def _mm_body(x_ref, w_ref, o_ref):
    o_ref[...] = jnp.dot(x_ref[...], w_ref[...],
                         preferred_element_type=jnp.float32)


def _matmul_x_w1(x, w1):
    r = 400
    return pl.pallas_call(
        _mm_body,
        grid=(N_NODES // r,),
        in_specs=[
            pl.BlockSpec((r, F_IN), lambda i: (i, 0)),
            pl.BlockSpec((F_IN, HID), lambda i: (0, 0)),
        ],
        out_specs=pl.BlockSpec((r, HID), lambda i: (i, 0)),
        out_shape=jax.ShapeDtypeStruct((N_NODES, HID), jnp.float32),
    )(x, w1)


def _mid_body(p0_ref, p1_ref, w2_ref, b1_ref, o_ref):
    h = jnp.maximum(p0_ref[...] + p1_ref[...] + b1_ref[...], 0.0)
    o_ref[...] = jnp.dot(h, w2_ref[...], preferred_element_type=jnp.float32)


def _middle(p0, p1, w2p, b1):
    r = 736
    return pl.pallas_call(
        _mid_body,
        grid=(N_PAD // r,),
        in_specs=[
            pl.BlockSpec((r, HID), lambda i: (i, 0)),
            pl.BlockSpec((r, HID), lambda i: (i, 0)),
            pl.BlockSpec((HID, 8), lambda i: (0, 0)),
            pl.BlockSpec((1, HID), lambda i: (0, 0)),
        ],
        out_specs=pl.BlockSpec((r, 8), lambda i: (i, 0)),
        out_shape=jax.ShapeDtypeStruct((N_PAD, 8), jnp.float32),
    )(p0, p1, w2p, b1)


def _fin_body(q0_ref, q1_ref, b2_ref, o_ref):
    o_ref[...] = q0_ref[...] + q1_ref[...] + b2_ref[...]


def _final(q0, q1, b2p):
    r = 736
    return pl.pallas_call(
        _fin_body,
        grid=(N_PAD // r,),
        in_specs=[
            pl.BlockSpec((r, 8), lambda i: (i, 0)),
            pl.BlockSpec((r, 8), lambda i: (i, 0)),
            pl.BlockSpec((1, 8), lambda i: (0, 0)),
        ],
        out_specs=pl.BlockSpec((r, 8), lambda i: (i, 0)),
        out_shape=jax.ShapeDtypeStruct((N_PAD, 8), jnp.float32),
    )(q0, q1, b2p)


def _make_agg(d, n_chunks):
    chunks_per_w = n_chunks // NW
    nbatches = chunks_per_w // BATCH
    assert chunks_per_w % BATCH == 0

    mesh = plsc.VectorSubcoreMesh(core_axis_name="c", subcore_axis_name="s")

    @functools.partial(
        pl.kernel,
        out_type=jax.ShapeDtypeStruct((2, N_PAD, d), jnp.float32),
        mesh=mesh,
        compiler_params=pltpu.CompilerParams(use_tc_tiling_on_sc=False),
        scratch_types=[
            pltpu.VMEM((BATCH, CHUNK), jnp.int32),
            pltpu.VMEM((BATCH, CHUNK), jnp.int32),
            pltpu.VMEM((CHUNK, d), jnp.float32),
            pltpu.VMEM_SHARED((N_PAD, d), jnp.float32),
        ],
    )
    def agg(h_hbm, src_hbm, dst_hbm, zeros_hbm, out_hbm, src_v, dst_v,
            rows_v, acc):
        cid = lax.axis_index("c")
        sid = lax.axis_index("s")
        wid = cid * 16 + sid

        pltpu.sync_copy(zeros_hbm,
                        acc.at[pl.ds(sid * ROWS_PER_TILE, ROWS_PER_TILE)])
        plsc.subcore_barrier()

        def batch_body(b, carry):
            cb = wid * chunks_per_w + b * BATCH
            pltpu.sync_copy(src_hbm.at[pl.ds(cb, BATCH)], src_v)
            pltpu.sync_copy(dst_hbm.at[pl.ds(cb, BATCH)], dst_v)

            def chunk_body(j, carry2):
                pltpu.sync_copy(h_hbm.at[src_v.at[j]], rows_v)
                pltpu.sync_copy(rows_v, acc.at[dst_v.at[j]], add=True)
                return carry2

            lax.fori_loop(0, BATCH, chunk_body, 0, unroll=False)
            return carry

        lax.fori_loop(0, nbatches, batch_body, 0, unroll=False)

        plsc.subcore_barrier()
        pltpu.sync_copy(acc.at[pl.ds(sid * ROWS_PER_TILE, ROWS_PER_TILE)],
                        out_hbm.at[cid, pl.ds(sid * ROWS_PER_TILE,
                                              ROWS_PER_TILE)])

    return agg


def kernel(x, edge_index, W1, b1, W2, b2):
    src = edge_index[0]
    dst = edge_index[1]
    e = src.shape[0]

    e_pad = NW * BATCH * CHUNK * (-(-e // (NW * BATCH * CHUNK)))
    pad = e_pad - e
    src_p = jnp.concatenate(
        [src.astype(jnp.int32), jnp.zeros((pad,), jnp.int32)])
    dst_p = jnp.concatenate(
        [dst.astype(jnp.int32), jnp.full((pad,), N_NODES, jnp.int32)])
    n_chunks = e_pad // CHUNK
    src2d = src_p.reshape(n_chunks, CHUNK)
    dst2d = dst_p.reshape(n_chunks, CHUNK)

    h1 = _matmul_x_w1(x, W1)

    zeros16 = jnp.zeros((ROWS_PER_TILE, HID), jnp.float32)
    agg16 = _make_agg(HID, n_chunks)
    part1 = agg16(h1, src2d, dst2d, zeros16)

    w2p = jnp.concatenate([W2, jnp.zeros((HID, 1), jnp.float32)], axis=1)
    h2 = _middle(part1[0], part1[1], w2p, b1.reshape(1, HID))

    zeros8 = jnp.zeros((ROWS_PER_TILE, 8), jnp.float32)
    agg8 = _make_agg(8, n_chunks)
    part2 = agg8(h2, src2d, dst2d, zeros8)

    b2p = jnp.concatenate([b2, jnp.zeros((1,), jnp.float32)]).reshape(1, 8)
    out = _final(part2[0], part2[1], b2p)
    return out[:N_NODES, :C_OUT]


# speedup vs baseline: 23.1243x; 1.4302x over previous
import functools

import jax
import jax.numpy as jnp
from jax import lax
from jax.experimental import pallas as pl
from jax.experimental.pallas import tpu as pltpu
from jax.experimental.pallas import tpu_sc as plsc

N_NODES = 50000
F_IN = 1433
HID = 16
C_OUT = 7

NW = 32
CHUNK = 128
BATCH = 40
N_PAD = 50048
ROWS_PER_TILE = N_PAD // 16


def _mm_body(x_ref, w_ref, o_ref):
    o_ref[...] = jnp.dot(x_ref[...], w_ref[...],
                         preferred_element_type=jnp.float32)


def _matmul_x_w1(x, w1):
    r = 400
    return pl.pallas_call(
        _mm_body,
        grid=(N_NODES // r,),
        in_specs=[
            pl.BlockSpec((r, F_IN), lambda i: (i, 0)),
            pl.BlockSpec((F_IN, HID), lambda i: (0, 0)),
        ],
        out_specs=pl.BlockSpec((r, HID), lambda i: (i, 0)),
        out_shape=jax.ShapeDtypeStruct((N_NODES, HID), jnp.float32),
    )(x, w1)


def _mid_body(p0_ref, p1_ref, w2_ref, b1_ref, o_ref):
    h = jnp.maximum(p0_ref[...] + p1_ref[...] + b1_ref[...], 0.0)
    o_ref[...] = jnp.dot(h, w2_ref[...], preferred_element_type=jnp.float32)


def _middle(p0, p1, w2p, b1):
    r = 736
    return pl.pallas_call(
        _mid_body,
        grid=(N_PAD // r,),
        in_specs=[
            pl.BlockSpec((r, HID), lambda i: (i, 0)),
            pl.BlockSpec((r, HID), lambda i: (i, 0)),
            pl.BlockSpec((HID, 8), lambda i: (0, 0)),
            pl.BlockSpec((1, HID), lambda i: (0, 0)),
        ],
        out_specs=pl.BlockSpec((r, 8), lambda i: (i, 0)),
        out_shape=jax.ShapeDtypeStruct((N_PAD, 8), jnp.float32),
    )(p0, p1, w2p, b1)


def _fin_body(q0_ref, q1_ref, b2_ref, o_ref):
    o_ref[...] = q0_ref[...] + q1_ref[...] + b2_ref[...]


def _final(q0, q1, b2p):
    r = 736
    return pl.pallas_call(
        _fin_body,
        grid=(N_PAD // r,),
        in_specs=[
            pl.BlockSpec((r, 8), lambda i: (i, 0)),
            pl.BlockSpec((r, 8), lambda i: (i, 0)),
            pl.BlockSpec((1, 8), lambda i: (0, 0)),
        ],
        out_specs=pl.BlockSpec((r, 8), lambda i: (i, 0)),
        out_shape=jax.ShapeDtypeStruct((N_PAD, 8), jnp.float32),
    )(q0, q1, b2p)


def _make_agg(d, n_chunks):
    chunks_per_w = n_chunks // NW
    k = 16
    ng = chunks_per_w // k
    assert chunks_per_w % k == 0 and ng % 2 == 0

    mesh = plsc.VectorSubcoreMesh(core_axis_name="c", subcore_axis_name="s")

    @functools.partial(
        pl.kernel,
        out_type=jax.ShapeDtypeStruct((2, N_PAD, d), jnp.float32),
        mesh=mesh,
        compiler_params=pltpu.CompilerParams(use_tc_tiling_on_sc=False),
        scratch_types=[
            pltpu.VMEM((3, k, CHUNK), jnp.int32),
            pltpu.VMEM((3, k, CHUNK), jnp.int32),
            pltpu.VMEM((2, k * CHUNK, d), jnp.float32),
            pltpu.VMEM_SHARED((N_PAD, d), jnp.float32),
            pltpu.SemaphoreType.DMA,
            pltpu.SemaphoreType.DMA,
            pltpu.SemaphoreType.DMA,
            pltpu.SemaphoreType.DMA,
            pltpu.SemaphoreType.DMA,
        ],
    )
    def agg(h_hbm, src_hbm, dst_hbm, zeros_hbm, out_hbm, src_v, dst_v,
            rows_v, acc, isem, gsem0, gsem1, ssem0, ssem1):
        cid = lax.axis_index("c")
        sid = lax.axis_index("s")
        wid = cid * 16 + sid
        base = wid * chunks_per_w
        gsem = (gsem0, gsem1)
        ssem = (ssem0, ssem1)

        pltpu.sync_copy(zeros_hbm,
                        acc.at[pl.ds(sid * ROWS_PER_TILE, ROWS_PER_TILE)])
        plsc.subcore_barrier()

        def idx_slot(g):
            return lax.rem(g, 3)

        def idx_pair(g):
            q = idx_slot(g)
            cb = base + g * k
            return (pltpu.make_async_copy(src_hbm.at[pl.ds(cb, k)],
                                          src_v.at[q], isem),
                    pltpu.make_async_copy(dst_hbm.at[pl.ds(cb, k)],
                                          dst_v.at[q], isem))

        def idx_start(g):
            for c in idx_pair(g):
                c.start()

        def idx_wait(g):
            for c in idx_pair(g):
                c.wait()

        def gath_copies(g, p):
            q = idx_slot(g)
            return [pltpu.make_async_copy(
                        h_hbm.at[src_v.at[q, j]],
                        rows_v.at[p, pl.ds(j * CHUNK, CHUNK)], gsem[p])
                    for j in range(k)]

        def scat_copies(g, p):
            q = idx_slot(g)
            return [pltpu.make_async_copy(
                        rows_v.at[p, pl.ds(j * CHUNK, CHUNK)],
                        acc.at[dst_v.at[q, j]], ssem[p])
                    for j in range(k)]

        def do_group(g, p):
            pp = 1 - p
            idx_wait(g)

            @pl.when(g >= 2)
            def _():
                for c in scat_copies(g - 2, p):
                    c.wait()

            for c in gath_copies(g, p):
                c.start()

            @pl.when(g + 1 < ng)
            def _():
                idx_start(g + 1)

            @pl.when(g >= 1)
            def _():
                for c in gath_copies(g - 1, pp):
                    c.wait()
                for c in scat_copies(g - 1, pp):
                    c.start(add=True)

        idx_start(0)

        def pair_body(t, carry):
            do_group(2 * t, 0)
            do_group(2 * t + 1, 1)
            return carry

        lax.fori_loop(0, ng // 2, pair_body, 0, unroll=False)

        for c in gath_copies(ng - 1, 1):
            c.wait()
        for c in scat_copies(ng - 1, 1):
            c.start(add=True)
        for c in scat_copies(ng - 2, 0):
            c.wait()
        for c in scat_copies(ng - 1, 1):
            c.wait()

        plsc.subcore_barrier()
        pltpu.sync_copy(acc.at[pl.ds(sid * ROWS_PER_TILE, ROWS_PER_TILE)],
                        out_hbm.at[cid, pl.ds(sid * ROWS_PER_TILE,
                                              ROWS_PER_TILE)])

    return agg


def kernel(x, edge_index, W1, b1, W2, b2):
    src = edge_index[0]
    dst = edge_index[1]
    e = src.shape[0]

    e_pad = NW * BATCH * CHUNK * (-(-e // (NW * BATCH * CHUNK)))
    pad = e_pad - e
    src_p = jnp.concatenate(
        [src.astype(jnp.int32), jnp.zeros((pad,), jnp.int32)])
    dst_p = jnp.concatenate(
        [dst.astype(jnp.int32), jnp.full((pad,), N_NODES, jnp.int32)])
    n_chunks = e_pad // CHUNK
    src2d = src_p.reshape(n_chunks, CHUNK)
    dst2d = dst_p.reshape(n_chunks, CHUNK)

    h1 = _matmul_x_w1(x, W1)

    zeros16 = jnp.zeros((ROWS_PER_TILE, HID), jnp.float32)
    agg16 = _make_agg(HID, n_chunks)
    part1 = agg16(h1, src2d, dst2d, zeros16)

    w2p = jnp.concatenate([W2, jnp.zeros((HID, 1), jnp.float32)], axis=1)
    h2 = _middle(part1[0], part1[1], w2p, b1.reshape(1, HID))

    zeros8 = jnp.zeros((ROWS_PER_TILE, 8), jnp.float32)
    agg8 = _make_agg(8, n_chunks)
    part2 = agg8(h2, src2d, dst2d, zeros8)

    b2p = jnp.concatenate([b2, jnp.zeros((1,), jnp.float32)]).reshape(1, 8)
    out = _final(part2[0], part2[1], b2p)
    return out[:N_NODES, :C_OUT]


# speedup vs baseline: 33.9310x; 1.4673x over previous
import functools

import jax
import jax.numpy as jnp
from jax import lax
from jax.experimental import pallas as pl
from jax.experimental.pallas import tpu as pltpu
from jax.experimental.pallas import tpu_sc as plsc

N_NODES = 50000
F_IN = 1433
HID = 16
C_OUT = 7

NW = 32
CHUNK = 128
N_PAD = 50176
ROWS_PER_TILE = N_PAD // 16


def _mm_body(xt_ref, w_ref, o_ref):
    o_ref[...] = lax.dot_general(
        xt_ref[...], w_ref[...], (((0,), (0,)), ((), ())),
        preferred_element_type=jnp.float32)


def _matmul_x_w1(xt, w1):
    r = 512
    return pl.pallas_call(
        _mm_body,
        grid=(N_PAD // r,),
        in_specs=[
            pl.BlockSpec((F_IN, r), lambda i: (0, i)),
            pl.BlockSpec((F_IN, HID), lambda i: (0, 0)),
        ],
        out_specs=pl.BlockSpec((r, HID), lambda i: (i, 0)),
        out_shape=jax.ShapeDtypeStruct((N_PAD, HID), jnp.float32),
    )(xt, w1)


def _mid_body(p0_ref, p1_ref, w2_ref, b1_ref, o_ref):
    h = jnp.maximum(p0_ref[...] + p1_ref[...] + b1_ref[...], 0.0)
    o_ref[...] = jnp.dot(h, w2_ref[...], preferred_element_type=jnp.float32)


def _middle(p0, p1, w2p, b1):
    r = 6272
    return pl.pallas_call(
        _mid_body,
        grid=(N_PAD // r,),
        in_specs=[
            pl.BlockSpec((r, HID), lambda i: (i, 0)),
            pl.BlockSpec((r, HID), lambda i: (i, 0)),
            pl.BlockSpec((HID, 8), lambda i: (0, 0)),
            pl.BlockSpec((1, HID), lambda i: (0, 0)),
        ],
        out_specs=pl.BlockSpec((r, 8), lambda i: (i, 0)),
        out_shape=jax.ShapeDtypeStruct((N_PAD, 8), jnp.float32),
    )(p0, p1, w2p, b1)


def _fin_body(q0_ref, q1_ref, b2_ref, o_ref):
    s = q0_ref[...] + q1_ref[...] + b2_ref[...]
    o_ref[...] = s.T[:C_OUT]


def _final(q0, q1, b2p):
    r = 6272
    return pl.pallas_call(
        _fin_body,
        grid=(N_PAD // r,),
        in_specs=[
            pl.BlockSpec((r, 8), lambda i: (i, 0)),
            pl.BlockSpec((r, 8), lambda i: (i, 0)),
            pl.BlockSpec((1, 8), lambda i: (0, 0)),
        ],
        out_specs=pl.BlockSpec((C_OUT, r), lambda i: (0, i)),
        out_shape=jax.ShapeDtypeStruct((C_OUT, N_PAD), jnp.float32),
    )(q0, q1, b2p)


def _make_agg(d, n_chunks, spmem_cache):
    chunks_per_w = n_chunks // NW
    k = 16
    ng = chunks_per_w // k
    assert chunks_per_w % k == 0 and ng % 2 == 0

    mesh = plsc.VectorSubcoreMesh(core_axis_name="c", subcore_axis_name="s")

    @functools.partial(
        pl.kernel,
        out_type=(jax.ShapeDtypeStruct((N_PAD, d), jnp.float32),
                  jax.ShapeDtypeStruct((N_PAD, d), jnp.float32)),
        mesh=mesh,
        compiler_params=pltpu.CompilerParams(use_tc_tiling_on_sc=False),
        scratch_types=[
            pltpu.VMEM((3, k, CHUNK), jnp.int32),
            pltpu.VMEM((3, k, CHUNK), jnp.int32),
            pltpu.VMEM((2, k * CHUNK, d), jnp.float32),
        ] + ([pltpu.VMEM_SHARED((N_PAD, d), jnp.float32)] if spmem_cache
             else []) + [
            pltpu.VMEM_SHARED((N_PAD, d), jnp.float32),
            pltpu.SemaphoreType.DMA,
            pltpu.SemaphoreType.DMA,
            pltpu.SemaphoreType.DMA,
            pltpu.SemaphoreType.DMA,
            pltpu.SemaphoreType.DMA,
        ],
    )
    def agg(h_hbm, src_hbm, dst_hbm, zeros_hbm, out0_hbm, out1_hbm,
            src_v, dst_v, rows_v, *rest):
        if spmem_cache:
            hcache, acc, isem, gsem0, gsem1, ssem0, ssem1 = rest
        else:
            acc, isem, gsem0, gsem1, ssem0, ssem1 = rest
            hcache = None
        cid = lax.axis_index("c")
        sid = lax.axis_index("s")
        wid = cid * 16 + sid
        base = wid * chunks_per_w
        gsem = (gsem0, gsem1)
        ssem = (ssem0, ssem1)
        tile_rows = pl.ds(sid * ROWS_PER_TILE, ROWS_PER_TILE)

        if spmem_cache:
            pltpu.sync_copy(h_hbm.at[tile_rows], hcache.at[tile_rows])
        pltpu.sync_copy(zeros_hbm, acc.at[tile_rows])
        plsc.subcore_barrier()
        h_src = hcache if spmem_cache else h_hbm

        def idx_slot(g):
            return lax.rem(g, 3)

        def idx_pair(g):
            q = idx_slot(g)
            cb = base + g * k
            return (pltpu.make_async_copy(src_hbm.at[pl.ds(cb, k)],
                                          src_v.at[q], isem),
                    pltpu.make_async_copy(dst_hbm.at[pl.ds(cb, k)],
                                          dst_v.at[q], isem))

        def idx_start(g):
            for c in idx_pair(g):
                c.start()

        def idx_wait(g):
            for c in idx_pair(g):
                c.wait()

        def gath_copies(g, p):
            q = idx_slot(g)
            return [pltpu.make_async_copy(
                        h_src.at[src_v.at[q, j]],
                        rows_v.at[p, pl.ds(j * CHUNK, CHUNK)], gsem[p])
                    for j in range(k)]

        def scat_copies(g, p):
            q = idx_slot(g)
            return [pltpu.make_async_copy(
                        rows_v.at[p, pl.ds(j * CHUNK, CHUNK)],
                        acc.at[dst_v.at[q, j]], ssem[p])
                    for j in range(k)]

        def do_group(g, p):
            pp = 1 - p
            idx_wait(g)

            @pl.when(g >= 2)
            def _():
                for c in scat_copies(g - 2, p):
                    c.wait()

            for c in gath_copies(g, p):
                c.start()

            @pl.when(g + 1 < ng)
            def _():
                idx_start(g + 1)

            @pl.when(g >= 1)
            def _():
                for c in gath_copies(g - 1, pp):
                    c.wait()
                for c in scat_copies(g - 1, pp):
                    c.start(add=True)

        idx_start(0)

        def pair_body(t, carry):
            do_group(2 * t, 0)
            do_group(2 * t + 1, 1)
            return carry

        lax.fori_loop(0, ng // 2, pair_body, 0, unroll=False)

        for c in gath_copies(ng - 1, 1):
            c.wait()
        for c in scat_copies(ng - 1, 1):
            c.start(add=True)
        for c in scat_copies(ng - 2, 0):
            c.wait()
        for c in scat_copies(ng - 1, 1):
            c.wait()

        plsc.subcore_barrier()

        @pl.when(cid == 0)
        def _():
            pltpu.sync_copy(acc.at[tile_rows], out0_hbm.at[tile_rows])

        @pl.when(cid == 1)
        def _():
            pltpu.sync_copy(acc.at[tile_rows], out1_hbm.at[tile_rows])

    return agg


def kernel(x, edge_index, W1, b1, W2, b2):
    src = edge_index[0]
    dst = edge_index[1]
    e = src.shape[0]

    unit = NW * 16 * CHUNK * 2
    e_pad = unit * (-(-e // unit))
    pad = e_pad - e
    src_p = jnp.concatenate(
        [src.astype(jnp.int32), jnp.zeros((pad,), jnp.int32)])
    dst_p = jnp.concatenate(
        [dst.astype(jnp.int32), jnp.full((pad,), N_NODES, jnp.int32)])
    n_chunks = e_pad // CHUNK
    src2d = src_p.reshape(n_chunks, CHUNK)
    dst2d = dst_p.reshape(n_chunks, CHUNK)

    h1 = _matmul_x_w1(x.T, W1)

    zeros16 = jnp.zeros((ROWS_PER_TILE, HID), jnp.float32)
    agg16 = _make_agg(HID, n_chunks, spmem_cache=False)
    p1a, p1b = agg16(h1, src2d, dst2d, zeros16)

    w2p = jnp.concatenate([W2, jnp.zeros((HID, 1), jnp.float32)], axis=1)
    h2 = _middle(p1a, p1b, w2p, b1.reshape(1, HID))

    zeros8 = jnp.zeros((ROWS_PER_TILE, 8), jnp.float32)
    agg8 = _make_agg(8, n_chunks, spmem_cache=True)
    p2a, p2b = agg8(h2, src2d, dst2d, zeros8)

    b2p = jnp.concatenate([b2, jnp.zeros((1,), jnp.float32)]).reshape(1, 8)
    out_t = _final(p2a, p2b, b2p)
    return out_t[:, :N_NODES].T


# speedup vs baseline: 54.8926x; 1.6178x over previous
import functools

import jax
import jax.numpy as jnp
from jax import lax
from jax.experimental import pallas as pl
from jax.experimental.pallas import tpu as pltpu
from jax.experimental.pallas import tpu_sc as plsc

N_NODES = 50000
F_IN = 1433
HID = 16
C_OUT = 7

NW = 32
CHUNK = 128
N_PAD = 50176
ROWS_PER_TILE = N_PAD // 16


def _mm_body(xt_ref, w_ref, oa_ref, ob_ref):
    h = lax.dot_general(
        xt_ref[...], w_ref[...], (((0,), (0,)), ((), ())),
        preferred_element_type=jnp.float32)
    oa_ref[...] = h[:, :8]
    ob_ref[...] = h[:, 8:]


def _matmul_x_w1(xt, w1):
    r = 512
    return pl.pallas_call(
        _mm_body,
        grid=(N_PAD // r,),
        in_specs=[
            pl.BlockSpec((F_IN, r), lambda i: (0, i)),
            pl.BlockSpec((F_IN, HID), lambda i: (0, 0)),
        ],
        out_specs=[pl.BlockSpec((r, 8), lambda i: (i, 0)),
                   pl.BlockSpec((r, 8), lambda i: (i, 0))],
        out_shape=(jax.ShapeDtypeStruct((N_PAD, 8), jnp.float32),
                   jax.ShapeDtypeStruct((N_PAD, 8), jnp.float32)),
    )(xt, w1)


def _mid_body(p0_ref, p1_ref, w2_ref, b1_ref, o_ref):
    h = jnp.concatenate([p0_ref[...], p1_ref[...]], axis=1)
    h = jnp.maximum(h + b1_ref[...], 0.0)
    o_ref[...] = jnp.dot(h, w2_ref[...], preferred_element_type=jnp.float32)


def _middle(p0, p1, w2p, b1):
    r = 6272
    return pl.pallas_call(
        _mid_body,
        grid=(N_PAD // r,),
        in_specs=[
            pl.BlockSpec((r, 8), lambda i: (i, 0)),
            pl.BlockSpec((r, 8), lambda i: (i, 0)),
            pl.BlockSpec((HID, 8), lambda i: (0, 0)),
            pl.BlockSpec((1, HID), lambda i: (0, 0)),
        ],
        out_specs=pl.BlockSpec((r, 8), lambda i: (i, 0)),
        out_shape=jax.ShapeDtypeStruct((N_PAD, 8), jnp.float32),
    )(p0, p1, w2p, b1)


def _fin_body(q0_ref, q1_ref, b2_ref, o_ref):
    s = q0_ref[...] + q1_ref[...] + b2_ref[...]
    o_ref[...] = s.T[:C_OUT]


def _final(q0, q1, b2p):
    r = 6272
    return pl.pallas_call(
        _fin_body,
        grid=(N_PAD // r,),
        in_specs=[
            pl.BlockSpec((r, 8), lambda i: (i, 0)),
            pl.BlockSpec((r, 8), lambda i: (i, 0)),
            pl.BlockSpec((1, 8), lambda i: (0, 0)),
        ],
        out_specs=pl.BlockSpec((C_OUT, r), lambda i: (0, i)),
        out_shape=jax.ShapeDtypeStruct((C_OUT, N_PAD), jnp.float32),
    )(q0, q1, b2p)


def _make_agg(d, n_chunks, spmem_cache, col_split=False):
    chunks_per_w = n_chunks // (16 if col_split else NW)
    k = 16
    ng = chunks_per_w // k
    assert chunks_per_w % k == 0 and ng % 2 == 0

    mesh = plsc.VectorSubcoreMesh(core_axis_name="c", subcore_axis_name="s")

    @functools.partial(
        pl.kernel,
        out_type=(jax.ShapeDtypeStruct((N_PAD, d), jnp.float32),
                  jax.ShapeDtypeStruct((N_PAD, d), jnp.float32)),
        mesh=mesh,
        compiler_params=pltpu.CompilerParams(use_tc_tiling_on_sc=False),
        scratch_types=[
            pltpu.VMEM((3, k, CHUNK), jnp.int32),
            pltpu.VMEM((3, k, CHUNK), jnp.int32),
            pltpu.VMEM((2, k * CHUNK, d), jnp.float32),
        ] + ([pltpu.VMEM_SHARED((N_PAD, d), jnp.float32)] if spmem_cache
             else []) + [
            pltpu.VMEM_SHARED((N_PAD, d), jnp.float32),
            pltpu.SemaphoreType.DMA,
            pltpu.SemaphoreType.DMA,
            pltpu.SemaphoreType.DMA,
            pltpu.SemaphoreType.DMA,
            pltpu.SemaphoreType.DMA,
        ],
    )
    def agg(h_hbm, h2_hbm, src_hbm, dst_hbm, zeros_hbm, out0_hbm, out1_hbm,
            src_v, dst_v, rows_v, *rest):
        if spmem_cache:
            hcache, acc, isem, gsem0, gsem1, ssem0, ssem1 = rest
        else:
            acc, isem, gsem0, gsem1, ssem0, ssem1 = rest
            hcache = None
        cid = lax.axis_index("c")
        sid = lax.axis_index("s")
        wid = sid if col_split else cid * 16 + sid
        base = wid * chunks_per_w
        gsem = (gsem0, gsem1)
        ssem = (ssem0, ssem1)
        tile_rows = pl.ds(sid * ROWS_PER_TILE, ROWS_PER_TILE)

        if col_split:
            @pl.when(cid == 0)
            def _():
                pltpu.sync_copy(h_hbm.at[tile_rows], hcache.at[tile_rows])

            @pl.when(cid == 1)
            def _():
                pltpu.sync_copy(h2_hbm.at[tile_rows], hcache.at[tile_rows])
        elif spmem_cache:
            pltpu.sync_copy(h_hbm.at[tile_rows], hcache.at[tile_rows])
        pltpu.sync_copy(zeros_hbm, acc.at[tile_rows])
        plsc.subcore_barrier()
        h_src = hcache if spmem_cache else h_hbm

        def idx_slot(g):
            return lax.rem(g, 3)

        def idx_pair(g):
            q = idx_slot(g)
            cb = base + g * k
            return (pltpu.make_async_copy(src_hbm.at[pl.ds(cb, k)],
                                          src_v.at[q], isem),
                    pltpu.make_async_copy(dst_hbm.at[pl.ds(cb, k)],
                                          dst_v.at[q], isem))

        def idx_start(g):
            for c in idx_pair(g):
                c.start()

        def idx_wait(g):
            for c in idx_pair(g):
                c.wait()

        def gath_copies(g, p):
            q = idx_slot(g)
            return [pltpu.make_async_copy(
                        h_src.at[src_v.at[q, j]],
                        rows_v.at[p, pl.ds(j * CHUNK, CHUNK)], gsem[p])
                    for j in range(k)]

        def scat_copies(g, p):
            q = idx_slot(g)
            return [pltpu.make_async_copy(
                        rows_v.at[p, pl.ds(j * CHUNK, CHUNK)],
                        acc.at[dst_v.at[q, j]], ssem[p])
                    for j in range(k)]

        def do_group(g, p):
            pp = 1 - p
            idx_wait(g)

            @pl.when(g >= 2)
            def _():
                for c in scat_copies(g - 2, p):
                    c.wait()

            for c in gath_copies(g, p):
                c.start()

            @pl.when(g + 1 < ng)
            def _():
                idx_start(g + 1)

            @pl.when(g >= 1)
            def _():
                for c in gath_copies(g - 1, pp):
                    c.wait()
                for c in scat_copies(g - 1, pp):
                    c.start(add=True)

        idx_start(0)

        def pair_body(t, carry):
            do_group(2 * t, 0)
            do_group(2 * t + 1, 1)
            return carry

        lax.fori_loop(0, ng // 2, pair_body, 0, unroll=False)

        for c in gath_copies(ng - 1, 1):
            c.wait()
        for c in scat_copies(ng - 1, 1):
            c.start(add=True)
        for c in scat_copies(ng - 2, 0):
            c.wait()
        for c in scat_copies(ng - 1, 1):
            c.wait()

        plsc.subcore_barrier()

        @pl.when(cid == 0)
        def _():
            pltpu.sync_copy(acc.at[tile_rows], out0_hbm.at[tile_rows])

        @pl.when(cid == 1)
        def _():
            pltpu.sync_copy(acc.at[tile_rows], out1_hbm.at[tile_rows])

    return agg


def kernel(x, edge_index, W1, b1, W2, b2):
    src = edge_index[0]
    dst = edge_index[1]
    e = src.shape[0]

    unit = NW * 16 * CHUNK * 2
    e_pad = unit * (-(-e // unit))
    pad = e_pad - e
    src_p = jnp.concatenate(
        [src.astype(jnp.int32), jnp.zeros((pad,), jnp.int32)])
    dst_p = jnp.concatenate(
        [dst.astype(jnp.int32), jnp.full((pad,), N_NODES, jnp.int32)])
    n_chunks = e_pad // CHUNK
    src2d = src_p.reshape(n_chunks, CHUNK)
    dst2d = dst_p.reshape(n_chunks, CHUNK)

    h1a, h1b = _matmul_x_w1(x.T, W1)

    zeros8 = jnp.zeros((ROWS_PER_TILE, 8), jnp.float32)
    agg_l1 = _make_agg(8, n_chunks, spmem_cache=True, col_split=True)
    p1a, p1b = agg_l1(h1a, h1b, src2d, dst2d, zeros8)

    w2p = jnp.concatenate([W2, jnp.zeros((HID, 1), jnp.float32)], axis=1)
    h2 = _middle(p1a, p1b, w2p, b1.reshape(1, HID))

    agg8 = _make_agg(8, n_chunks, spmem_cache=True)
    p2a, p2b = agg8(h2, h2, src2d, dst2d, zeros8)

    b2p = jnp.concatenate([b2, jnp.zeros((1,), jnp.float32)]).reshape(1, 8)
    out_t = _final(p2a, p2b, b2p)
    return out_t[:, :N_NODES].T


# speedup vs baseline: 63.6601x; 1.1597x over previous
import functools

import jax
import jax.numpy as jnp
from jax import lax
from jax.experimental import pallas as pl
from jax.experimental.pallas import tpu as pltpu
from jax.experimental.pallas import tpu_sc as plsc

N_NODES = 50000
F_IN = 1433
HID = 16
C_OUT = 7

NW = 32
CHUNK = 128
N_PAD = 50176
ROWS_PER_TILE = N_PAD // 16


def _mm_body(xt_ref, w_ref, oa_ref, ob_ref):
    h = lax.dot_general(
        xt_ref[...], w_ref[...], (((0,), (0,)), ((), ())),
        preferred_element_type=jnp.float32)
    oa_ref[...] = h[:, :8]
    ob_ref[...] = h[:, 8:]


def _matmul_x_w1(xt, w1):
    r = 1024
    return pl.pallas_call(
        _mm_body,
        grid=(N_PAD // r,),
        in_specs=[
            pl.BlockSpec((F_IN, r), lambda i: (0, i)),
            pl.BlockSpec((F_IN, HID), lambda i: (0, 0)),
        ],
        out_specs=[pl.BlockSpec((r, 8), lambda i: (i, 0)),
                   pl.BlockSpec((r, 8), lambda i: (i, 0))],
        out_shape=(jax.ShapeDtypeStruct((N_PAD, 8), jnp.float32),
                   jax.ShapeDtypeStruct((N_PAD, 8), jnp.float32)),
    )(xt, w1)


def _mid_body(p0_ref, p1_ref, wa_ref, wb_ref, b1a_ref, b1b_ref, o_ref):
    ha = jnp.maximum(p0_ref[...] + b1a_ref[...], 0.0)
    hb = jnp.maximum(p1_ref[...] + b1b_ref[...], 0.0)
    o_ref[...] = (
        jnp.dot(ha, wa_ref[...], preferred_element_type=jnp.float32)
        + jnp.dot(hb, wb_ref[...], preferred_element_type=jnp.float32))


def _middle(p0, p1, wa, wb, b1a, b1b):
    r = 392
    m = N_PAD // 16
    return pl.pallas_call(
        _mid_body,
        grid=(m // r,),
        in_specs=[
            pl.BlockSpec((r, 128), lambda i: (i, 0)),
            pl.BlockSpec((r, 128), lambda i: (i, 0)),
            pl.BlockSpec((128, 128), lambda i: (0, 0)),
            pl.BlockSpec((128, 128), lambda i: (0, 0)),
            pl.BlockSpec((1, 128), lambda i: (0, 0)),
            pl.BlockSpec((1, 128), lambda i: (0, 0)),
        ],
        out_specs=pl.BlockSpec((r, 128), lambda i: (i, 0)),
        out_shape=jax.ShapeDtypeStruct((m, 128), jnp.float32),
    )(p0, p1, wa, wb, b1a, b1b)


def _fin_body(q0_ref, q1_ref, b2_ref, o_ref):
    o_ref[...] = q0_ref[...] + q1_ref[...] + b2_ref[...]


def _final(q0, q1, b2v):
    r = 392
    m = N_PAD // 16
    return pl.pallas_call(
        _fin_body,
        grid=(m // r,),
        in_specs=[
            pl.BlockSpec((r, 128), lambda i: (i, 0)),
            pl.BlockSpec((r, 128), lambda i: (i, 0)),
            pl.BlockSpec((1, 128), lambda i: (0, 0)),
        ],
        out_specs=pl.BlockSpec((r, 128), lambda i: (i, 0)),
        out_shape=jax.ShapeDtypeStruct((N_PAD // 16, 128), jnp.float32),
    )(q0, q1, b2v)


def _make_agg(d, n_chunks, spmem_cache, col_split=False):
    chunks_per_w = n_chunks // (16 if col_split else NW)
    k = 16
    ng = chunks_per_w // k
    assert chunks_per_w % k == 0 and ng % 2 == 0

    mesh = plsc.VectorSubcoreMesh(core_axis_name="c", subcore_axis_name="s")

    @functools.partial(
        pl.kernel,
        out_type=(jax.ShapeDtypeStruct((N_PAD, d), jnp.float32),
                  jax.ShapeDtypeStruct((N_PAD, d), jnp.float32)),
        mesh=mesh,
        compiler_params=pltpu.CompilerParams(use_tc_tiling_on_sc=False),
        scratch_types=[
            pltpu.VMEM((3, k, CHUNK), jnp.int32),
            pltpu.VMEM((3, k, CHUNK), jnp.int32),
            pltpu.VMEM((2, k * CHUNK, d), jnp.float32),
        ] + ([pltpu.VMEM_SHARED((N_PAD, d), jnp.float32)] if spmem_cache
             else []) + [
            pltpu.VMEM_SHARED((N_PAD, d), jnp.float32),
            pltpu.SemaphoreType.DMA,
            pltpu.SemaphoreType.DMA,
            pltpu.SemaphoreType.DMA,
            pltpu.SemaphoreType.DMA,
            pltpu.SemaphoreType.DMA,
        ],
    )
    def agg(h_hbm, h2_hbm, src_hbm, dst_hbm, zeros_hbm, out0_hbm, out1_hbm,
            src_v, dst_v, rows_v, *rest):
        if spmem_cache:
            hcache, acc, isem, gsem0, gsem1, ssem0, ssem1 = rest
        else:
            acc, isem, gsem0, gsem1, ssem0, ssem1 = rest
            hcache = None
        cid = lax.axis_index("c")
        sid = lax.axis_index("s")
        wid = sid if col_split else cid * 16 + sid
        base = wid * chunks_per_w
        gsem = (gsem0, gsem1)
        ssem = (ssem0, ssem1)
        tile_rows = pl.ds(sid * ROWS_PER_TILE, ROWS_PER_TILE)

        if col_split:
            @pl.when(cid == 0)
            def _():
                pltpu.sync_copy(h_hbm.at[tile_rows], hcache.at[tile_rows])

            @pl.when(cid == 1)
            def _():
                pltpu.sync_copy(h2_hbm.at[tile_rows], hcache.at[tile_rows])
        elif spmem_cache:
            pltpu.sync_copy(h_hbm.at[tile_rows], hcache.at[tile_rows])
        pltpu.sync_copy(zeros_hbm, acc.at[tile_rows])
        plsc.subcore_barrier()
        h_src = hcache if spmem_cache else h_hbm

        def idx_slot(g):
            return lax.rem(g, 3)

        def idx_pair(g):
            q = idx_slot(g)
            cb = base + g * k
            return (pltpu.make_async_copy(src_hbm.at[pl.ds(cb, k)],
                                          src_v.at[q], isem),
                    pltpu.make_async_copy(dst_hbm.at[pl.ds(cb, k)],
                                          dst_v.at[q], isem))

        def idx_start(g):
            for c in idx_pair(g):
                c.start()

        def idx_wait(g):
            for c in idx_pair(g):
                c.wait()

        def gath_copies(g, p):
            q = idx_slot(g)
            return [pltpu.make_async_copy(
                        h_src.at[src_v.at[q, j]],
                        rows_v.at[p, pl.ds(j * CHUNK, CHUNK)], gsem[p])
                    for j in range(k)]

        def scat_copies(g, p):
            q = idx_slot(g)
            return [pltpu.make_async_copy(
                        rows_v.at[p, pl.ds(j * CHUNK, CHUNK)],
                        acc.at[dst_v.at[q, j]], ssem[p])
                    for j in range(k)]

        def do_group(g, p):
            pp = 1 - p
            idx_wait(g)

            @pl.when(g >= 2)
            def _():
                for c in scat_copies(g - 2, p):
                    c.wait()

            for c in gath_copies(g, p):
                c.start()

            @pl.when(g + 1 < ng)
            def _():
                idx_start(g + 1)

            @pl.when(g >= 1)
            def _():
                for c in gath_copies(g - 1, pp):
                    c.wait()
                for c in scat_copies(g - 1, pp):
                    c.start(add=True)

        idx_start(0)

        def pair_body(t, carry):
            do_group(2 * t, 0)
            do_group(2 * t + 1, 1)
            return carry

        lax.fori_loop(0, ng // 2, pair_body, 0, unroll=False)

        for c in gath_copies(ng - 1, 1):
            c.wait()
        for c in scat_copies(ng - 1, 1):
            c.start(add=True)
        for c in scat_copies(ng - 2, 0):
            c.wait()
        for c in scat_copies(ng - 1, 1):
            c.wait()

        plsc.subcore_barrier()

        @pl.when(cid == 0)
        def _():
            pltpu.sync_copy(acc.at[tile_rows], out0_hbm.at[tile_rows])

        @pl.when(cid == 1)
        def _():
            pltpu.sync_copy(acc.at[tile_rows], out1_hbm.at[tile_rows])

    return agg


def kernel(x, edge_index, W1, b1, W2, b2):
    src = edge_index[0]
    dst = edge_index[1]
    e = src.shape[0]

    unit = NW * 16 * CHUNK * 2
    e_pad = unit * (-(-e // unit))
    pad = e_pad - e
    src_p = jnp.concatenate(
        [src.astype(jnp.int32), jnp.zeros((pad,), jnp.int32)])
    dst_p = jnp.concatenate(
        [dst.astype(jnp.int32), jnp.full((pad,), N_NODES, jnp.int32)])
    n_chunks = e_pad // CHUNK
    src2d = src_p.reshape(n_chunks, CHUNK)
    dst2d = dst_p.reshape(n_chunks, CHUNK)

    h1a, h1b = _matmul_x_w1(x.T, W1)

    zeros8 = jnp.zeros((ROWS_PER_TILE, 8), jnp.float32)
    agg_l1 = _make_agg(8, n_chunks, spmem_cache=True, col_split=True)
    p1a, p1b = agg_l1(h1a, h1b, src2d, dst2d, zeros8)

    w2p = jnp.concatenate([W2, jnp.zeros((HID, 1), jnp.float32)], axis=1)
    eye16 = jnp.eye(16, dtype=jnp.float32)
    wa = jnp.kron(eye16, w2p[:8])
    wb = jnp.kron(eye16, w2p[8:])
    b1a = jnp.tile(b1[:8], 16).reshape(1, 128)
    b1b = jnp.tile(b1[8:], 16).reshape(1, 128)
    m = N_PAD // 16
    h2_pk = _middle(p1a.reshape(m, 128), p1b.reshape(m, 128),
                    wa, wb, b1a, b1b)
    h2 = h2_pk.reshape(N_PAD, 8)

    agg8 = _make_agg(8, n_chunks, spmem_cache=True)
    p2a, p2b = agg8(h2, h2, src2d, dst2d, zeros8)

    b2p = jnp.concatenate([b2, jnp.zeros((1,), jnp.float32)])
    b2v = jnp.tile(b2p, 16).reshape(1, 128)
    out_pk = _final(p2a.reshape(m, 128), p2b.reshape(m, 128), b2v)
    return out_pk.reshape(N_PAD, 8)[:N_NODES, :C_OUT]


# speedup vs baseline: 64.8915x; 1.0193x over previous
import functools

import jax
import jax.numpy as jnp
from jax import lax
from jax.experimental import pallas as pl
from jax.experimental.pallas import tpu as pltpu
from jax.experimental.pallas import tpu_sc as plsc

N_NODES = 50000
F_IN = 1433
HID = 16
C_OUT = 7

NW = 32
CHUNK = 128
N_PAD = 50176
ROWS_PER_TILE = N_PAD // 16


def _mm_body(xt_ref, w_ref, oa_ref, ob_ref):
    h = lax.dot_general(
        xt_ref[...], w_ref[...], (((0,), (0,)), ((), ())),
        preferred_element_type=jnp.float32)
    oa_ref[...] = h[:, :8]
    ob_ref[...] = h[:, 8:]


def _matmul_x_w1(xt, w1):
    r = 1024
    return pl.pallas_call(
        _mm_body,
        grid=(N_PAD // r,),
        in_specs=[
            pl.BlockSpec((F_IN, r), lambda i: (0, i)),
            pl.BlockSpec((F_IN, HID), lambda i: (0, 0)),
        ],
        out_specs=[pl.BlockSpec((r, 8), lambda i: (i, 0)),
                   pl.BlockSpec((r, 8), lambda i: (i, 0))],
        out_shape=(jax.ShapeDtypeStruct((N_PAD, 8), jnp.float32),
                   jax.ShapeDtypeStruct((N_PAD, 8), jnp.float32)),
    )(xt, w1)


def _mid_body(p0_ref, p1_ref, wa_ref, wb_ref, b1a_ref, b1b_ref, o_ref):
    ha = jnp.maximum(p0_ref[...] + b1a_ref[...], 0.0)
    hb = jnp.maximum(p1_ref[...] + b1b_ref[...], 0.0)
    o_ref[...] = (
        jnp.dot(ha, wa_ref[...], preferred_element_type=jnp.float32)
        + jnp.dot(hb, wb_ref[...], preferred_element_type=jnp.float32))


def _middle(p0, p1, wa, wb, b1a, b1b):
    r = 392
    m = N_PAD // 16
    return pl.pallas_call(
        _mid_body,
        grid=(m // r,),
        in_specs=[
            pl.BlockSpec((r, 128), lambda i: (i, 0)),
            pl.BlockSpec((r, 128), lambda i: (i, 0)),
            pl.BlockSpec((128, 128), lambda i: (0, 0)),
            pl.BlockSpec((128, 128), lambda i: (0, 0)),
            pl.BlockSpec((1, 128), lambda i: (0, 0)),
            pl.BlockSpec((1, 128), lambda i: (0, 0)),
        ],
        out_specs=pl.BlockSpec((r, 128), lambda i: (i, 0)),
        out_shape=jax.ShapeDtypeStruct((m, 128), jnp.float32),
    )(p0, p1, wa, wb, b1a, b1b)


def _fin_body(q0_ref, q1_ref, b2_ref, o_ref):
    o_ref[...] = q0_ref[...] + q1_ref[...] + b2_ref[...]


def _final(q0, q1, b2v):
    r = 392
    m = N_PAD // 16
    return pl.pallas_call(
        _fin_body,
        grid=(m // r,),
        in_specs=[
            pl.BlockSpec((r, 128), lambda i: (i, 0)),
            pl.BlockSpec((r, 128), lambda i: (i, 0)),
            pl.BlockSpec((1, 128), lambda i: (0, 0)),
        ],
        out_specs=pl.BlockSpec((r, 128), lambda i: (i, 0)),
        out_shape=jax.ShapeDtypeStruct((N_PAD // 16, 128), jnp.float32),
    )(q0, q1, b2v)


def _make_agg(d, n_chunks, mcw, tg, col_split, spmem_cache):
    k = 16
    workers = 16 if col_split else NW
    mg = mcw // k
    ng = mg + tg
    assert mcw % k == 0 and ng % 2 == 0 and tg in (1, 2)
    assert workers * mcw <= n_chunks <= workers * mcw + workers * tg * k

    mesh = plsc.VectorSubcoreMesh(core_axis_name="c", subcore_axis_name="s")

    @functools.partial(
        pl.kernel,
        out_type=(jax.ShapeDtypeStruct((N_PAD, d), jnp.float32),
                  jax.ShapeDtypeStruct((N_PAD, d), jnp.float32)),
        mesh=mesh,
        compiler_params=pltpu.CompilerParams(use_tc_tiling_on_sc=False),
        scratch_types=[
            pltpu.VMEM((3, k, 2, CHUNK), jnp.int32),
            pltpu.VMEM((2, k * CHUNK, d), jnp.float32),
        ] + ([pltpu.VMEM_SHARED((N_PAD, d), jnp.float32)] if spmem_cache
             else []) + [
            pltpu.VMEM_SHARED((N_PAD, d), jnp.float32),
            pltpu.SemaphoreType.DMA,
            pltpu.SemaphoreType.DMA,
            pltpu.SemaphoreType.DMA,
            pltpu.SemaphoreType.DMA,
            pltpu.SemaphoreType.DMA,
        ],
    )
    def agg(h_hbm, h2_hbm, e3_hbm, tail_hbm, zeros_hbm, out0_hbm, out1_hbm,
            idx_v, rows_v, *rest):
        if spmem_cache:
            hcache, acc, isem, gsem0, gsem1, ssem0, ssem1 = rest
        else:
            acc, isem, gsem0, gsem1, ssem0, ssem1 = rest
            hcache = None
        cid = lax.axis_index("c")
        sid = lax.axis_index("s")
        wid = sid if col_split else cid * 16 + sid
        gsem = (gsem0, gsem1)
        ssem = (ssem0, ssem1)
        tile_rows = pl.ds(sid * ROWS_PER_TILE, ROWS_PER_TILE)

        if col_split:
            @pl.when(cid == 0)
            def _():
                pltpu.sync_copy(h_hbm.at[tile_rows], hcache.at[tile_rows])

            @pl.when(cid == 1)
            def _():
                pltpu.sync_copy(h2_hbm.at[tile_rows], hcache.at[tile_rows])
        elif spmem_cache:
            pltpu.sync_copy(h_hbm.at[tile_rows], hcache.at[tile_rows])
        pltpu.sync_copy(zeros_hbm, acc.at[tile_rows])
        plsc.subcore_barrier()
        h_src = hcache if spmem_cache else h_hbm

        def loc(g):
            if isinstance(g, int) and g >= mg:
                return tail_hbm, wid * (tg * k) + (g - mg) * k
            return e3_hbm, wid * mcw + g * k

        def idx_slot(g):
            return lax.rem(g, 3) if not isinstance(g, int) else g % 3

        def idx_copy(g):
            ref, cb = loc(g)
            return pltpu.make_async_copy(ref.at[pl.ds(cb, k)],
                                         idx_v.at[idx_slot(g)], isem)

        def gath_copies(g, p):
            q = idx_slot(g)
            return [pltpu.make_async_copy(
                        h_src.at[idx_v.at[q, j, 0]],
                        rows_v.at[p, pl.ds(j * CHUNK, CHUNK)], gsem[p])
                    for j in range(k)]

        def scat_copies(g, p):
            q = idx_slot(g)
            return [pltpu.make_async_copy(
                        rows_v.at[p, pl.ds(j * CHUNK, CHUNK)],
                        acc.at[idx_v.at[q, j, 1]], ssem[p])
                    for j in range(k)]

        def do_group(g, p, prefetch):
            pp = 1 - p
            idx_copy(g).wait()

            @pl.when(g >= 2)
            def _():
                for c in scat_copies(g - 2, p):
                    c.wait()

            for c in gath_copies(g, p):
                c.start()

            prefetch()

            @pl.when(g >= 1)
            def _():
                for c in gath_copies(g - 1, pp):
                    c.wait()
                for c in scat_copies(g - 1, pp):
                    c.start(add=True)

        idx_copy(0).start()

        def pair_body(t, carry):
            g0 = 2 * t

            def pf0():
                idx_copy(g0 + 1).start()

            def pf1():
                @pl.when(g0 + 2 <= ng - 3)
                def _():
                    idx_copy(g0 + 2).start()

            do_group(g0, 0, pf0)
            do_group(g0 + 1, 1, pf1)
            return carry

        lax.fori_loop(0, (ng - 2) // 2, pair_body, 0, unroll=False)

        idx_copy(ng - 2).start()
        do_group(ng - 2, 0, lambda: idx_copy(ng - 1).start())
        do_group(ng - 1, 1, lambda: None)
        for c in gath_copies(ng - 1, 1):
            c.wait()
        for c in scat_copies(ng - 1, 1):
            c.start(add=True)
        for c in scat_copies(ng - 2, 0):
            c.wait()
        for c in scat_copies(ng - 1, 1):
            c.wait()

        plsc.subcore_barrier()

        @pl.when(cid == 0)
        def _():
            pltpu.sync_copy(acc.at[tile_rows], out0_hbm.at[tile_rows])

        @pl.when(cid == 1)
        def _():
            pltpu.sync_copy(acc.at[tile_rows], out1_hbm.at[tile_rows])

    return agg


def _tail_edges(src, dst, cov_chunks, tail_chunks):
    ts = src[cov_chunks * CHUNK:]
    td = dst[cov_chunks * CHUNK:]
    pad = tail_chunks * CHUNK - ts.shape[0]
    ts = jnp.concatenate([ts.astype(jnp.int32), jnp.zeros((pad,), jnp.int32)])
    td = jnp.concatenate([td.astype(jnp.int32),
                          jnp.full((pad,), N_NODES, jnp.int32)])
    return jnp.stack([ts.reshape(tail_chunks, CHUNK),
                      td.reshape(tail_chunks, CHUNK)], axis=1)


def kernel(x, edge_index, W1, b1, W2, b2):
    src = edge_index[0]
    dst = edge_index[1]
    e = src.shape[0]
    n_chunks = e // CHUNK
    assert e % CHUNK == 0

    e3 = jnp.swapaxes(
        edge_index.astype(jnp.int32).reshape(2, n_chunks, CHUNK), 0, 1)

    mcw1 = (n_chunks // 16) // 16 * 16
    mcw2 = (n_chunks // NW) // 32 * 32
    tail1 = _tail_edges(src, dst, 16 * mcw1, 16 * 16)
    tail2 = _tail_edges(src, dst, NW * mcw2, NW * 32)

    h1a, h1b = _matmul_x_w1(x.T, W1)

    zeros8 = jnp.zeros((ROWS_PER_TILE, 8), jnp.float32)
    agg_l1 = _make_agg(8, n_chunks, mcw1, 1, col_split=True,
                       spmem_cache=True)
    p1a, p1b = agg_l1(h1a, h1b, e3, tail1, zeros8)

    w2p = jnp.concatenate([W2, jnp.zeros((HID, 1), jnp.float32)], axis=1)
    eye16 = jnp.eye(16, dtype=jnp.float32)
    wa = jnp.kron(eye16, w2p[:8])
    wb = jnp.kron(eye16, w2p[8:])
    b1a = jnp.tile(b1[:8], 16).reshape(1, 128)
    b1b = jnp.tile(b1[8:], 16).reshape(1, 128)
    m = N_PAD // 16
    h2_pk = _middle(p1a.reshape(m, 128), p1b.reshape(m, 128),
                    wa, wb, b1a, b1b)
    h2 = h2_pk.reshape(N_PAD, 8)

    agg_l2 = _make_agg(8, n_chunks, mcw2, 2, col_split=False,
                       spmem_cache=True)
    p2a, p2b = agg_l2(h2, h2, e3, tail2, zeros8)

    b2p = jnp.concatenate([b2, jnp.zeros((1,), jnp.float32)])
    b2v = jnp.tile(b2p, 16).reshape(1, 128)
    out_pk = _final(p2a.reshape(m, 128), p2b.reshape(m, 128), b2v)
    return out_pk[:N_NODES // 16].reshape(N_NODES, 8)[:, :C_OUT]
